# Initial kernel scaffold; baseline (speedup 1.0000x reference)
import functools

import jax
import jax.numpy as jnp
from jax import lax
from jax.experimental import pallas as pl
from jax.experimental.pallas import tpu as pltpu
from jax.experimental.pallas import tpu_sc as plsc

D = 128
NC = 2
NS = 16
NW = NC * NS
CH = 128
NBUF = 4
DW = 16


def _ceil_to(a, m):
    return (a + m - 1) // m * m


def _make_sc_kernels(NP, K):
    RPT = NP // NS
    G = K // NBUF
    assert RPT % CH == 0 and K % NBUF == 0 and G >= 3
    mesh = plsc.VectorSubcoreMesh(core_axis_name="c", subcore_axis_name="s")
    zero16 = jnp.zeros((16,), jnp.float32)
    one16 = jnp.ones((16,), jnp.float32)

    def agg_body(hp, edge, out, acc, src_v, dst_v, rows_v,
                 gs0, gs1, gs2, gs3, ss0, ss1, ss2, ss3):
        gsems = (gs0, gs1, gs2, gs3)
        ssems = (ss0, ss1, ss2, ss3)
        c = lax.axis_index("c")
        s = lax.axis_index("s")
        base = (c * NS + s) * K
        pltpu.sync_copy(edge.at[0, pl.ds(base, K), :], src_v)
        pltpu.sync_copy(edge.at[1, pl.ds(base, K), :], dst_v)

        def zrow(r, carry):
            for kk in range(D // 16):
                rows_v[0, r, pl.ds(kk * 16, 16)] = zero16
            return carry
        lax.fori_loop(0, CH, zrow, 0)
        row0 = s * RPT
        for i in range(RPT // CH):
            pltpu.sync_copy(rows_v.at[0], acc.at[pl.ds(row0 + i * CH, CH), :])
        plsc.subcore_barrier()

        def gather(t, b):
            pltpu.async_copy(hp.at[src_v.at[t]], rows_v.at[b], gsems[b])

        def gather_wait(t, b):
            pltpu.make_async_copy(hp.at[src_v.at[t]], rows_v.at[b],
                                  gsems[b]).wait()

        def scatter(t, b):
            pltpu.async_copy(rows_v.at[b], acc.at[dst_v.at[t]], ssems[b],
                             add=True)

        def scatter_wait(b):
            pltpu.make_async_copy(rows_v.at[b], acc.at[dst_v.at[0]],
                                  ssems[b]).wait()

        for b in range(NBUF - 1):
            gather(b, b)

        for b in range(NBUF):
            gather_wait(b, b)
            scatter(b, b)
            bn = (b + NBUF - 1) % NBUF
            if b >= 1:
                scatter_wait(bn)
            gather(b + NBUF - 1, bn)

        def group(g, carry):
            for b in range(NBUF):
                t = g * NBUF + b
                gather_wait(t, b)
                scatter(t, b)
                bn = (b + NBUF - 1) % NBUF
                scatter_wait(bn)
                gather(t + NBUF - 1, bn)
            return carry
        lax.fori_loop(1, G - 1, group, 0)

        t0 = (G - 1) * NBUF
        for b in range(NBUF):
            gather_wait(t0 + b, b)
            scatter(t0 + b, b)
            if b == 0:
                scatter_wait(NBUF - 1)
                gather(t0 + NBUF - 1, NBUF - 1)

        for b in range(NBUF):
            scatter_wait(b)
        plsc.subcore_barrier()
        pltpu.sync_copy(acc.at[pl.ds(row0, RPT), :],
                        out.at[c, pl.ds(row0, RPT), :])

    agg = pl.kernel(
        agg_body,
        out_type=jax.ShapeDtypeStruct((NC, NP, D), jnp.float32),
        mesh=mesh,
        scratch_types=[
            pltpu.VMEM_SHARED((NP, D), jnp.float32),
            pltpu.VMEM((K, CH), jnp.int32),
            pltpu.VMEM((K, CH), jnp.int32),
            pltpu.VMEM((NBUF, CH, D), jnp.float32),
        ] + [pltpu.SemaphoreType.DMA] * (2 * NBUF),
    )

    ZR = 40
    assert RPT % ZR == 0

    def deg_body(edge, out, dacc, dst_v, ones_v, zb, sem):
        c = lax.axis_index("c")
        s = lax.axis_index("s")
        base = (c * NS + s) * K
        pltpu.sync_copy(edge.at[1, pl.ds(base, K), :], dst_v)

        def fill_one(r, carry):
            ones_v[r, :] = one16
            return carry
        lax.fori_loop(0, CH, fill_one, 0)

        def fill_zero(r, carry):
            zb[r, :] = zero16
            return carry
        lax.fori_loop(0, ZR, fill_zero, 0)
        for i in range(RPT // ZR):
            pltpu.sync_copy(zb, dacc.at[pl.ds(s * RPT + i * ZR, ZR), :])
        plsc.subcore_barrier()

        FAN = 8
        def grp(g, carry):
            for b in range(FAN):
                pltpu.async_copy(ones_v, dacc.at[dst_v.at[g * FAN + b]], sem,
                                 add=True)
            for b in range(FAN):
                pltpu.make_async_copy(ones_v, dacc.at[dst_v.at[0]],
                                      sem).wait()
            return carry
        lax.fori_loop(0, K // FAN, grp, 0)
        plsc.subcore_barrier()
        pltpu.sync_copy(dacc.at[pl.ds(s * RPT, RPT), :],
                        out.at[c, pl.ds(s * RPT, RPT), :])

    deg = pl.kernel(
        deg_body,
        out_type=jax.ShapeDtypeStruct((NC, NP, DW), jnp.float32),
        mesh=mesh,
        scratch_types=[
            pltpu.VMEM_SHARED((NP, DW), jnp.float32),
            pltpu.VMEM((K, CH), jnp.int32),
            pltpu.VMEM((CH, DW), jnp.float32),
            pltpu.VMEM((ZR, DW), jnp.float32),
            pltpu.SemaphoreType.DMA,
        ],
    )

    return agg, deg


def _dinv_block(deg_ref, blk, row0, n):
    degs = deg_ref[0, :, 0] + deg_ref[1, :, 0] + 1.0
    dinv = lax.rsqrt(degs)
    rows = lax.broadcasted_iota(jnp.int32, (blk,), 0) + row0
    return jnp.where(rows < n, dinv, 0.0)[:, None]


def _make_tc_kernels(N, NP):
    BLK = 1024
    assert NP % BLK == 0
    grid = NP // BLK

    def mm1_body(x_ref, w_ref, deg_ref, o_ref):
        i = pl.program_id(0)
        dinv = _dinv_block(deg_ref, BLK, i * BLK, N)
        o_ref[...] = jnp.dot(x_ref[...], w_ref[...],
                             preferred_element_type=jnp.float32) * dinv

    mm1 = pl.pallas_call(
        mm1_body,
        grid=(grid,),
        in_specs=[
            pl.BlockSpec((BLK, D), lambda i: (i, 0)),
            pl.BlockSpec((D, D), lambda i: (0, 0)),
            pl.BlockSpec((NC, BLK, DW), lambda i: (0, i, 0)),
        ],
        out_specs=pl.BlockSpec((BLK, D), lambda i: (i, 0)),
        out_shape=jax.ShapeDtypeStruct((NP, D), jnp.float32),
    )

    def mid_body(agg_ref, hp_ref, deg_ref, b_ref, w_ref, o_ref):
        i = pl.program_id(0)
        dinv = _dinv_block(deg_ref, BLK, i * BLK, N)
        u = (agg_ref[0] + agg_ref[1] + hp_ref[...]) * dinv + b_ref[...]
        nrm = jnp.sqrt(jnp.sum(u * u, axis=1, keepdims=True))
        v = jnp.maximum(u / jnp.maximum(nrm, 1e-12), 0.0)
        o_ref[...] = jnp.dot(v, w_ref[...],
                             preferred_element_type=jnp.float32) * dinv

    mid = pl.pallas_call(
        mid_body,
        grid=(grid,),
        in_specs=[
            pl.BlockSpec((NC, BLK, D), lambda i: (0, i, 0)),
            pl.BlockSpec((BLK, D), lambda i: (i, 0)),
            pl.BlockSpec((NC, BLK, DW), lambda i: (0, i, 0)),
            pl.BlockSpec((1, D), lambda i: (0, 0)),
            pl.BlockSpec((D, D), lambda i: (0, 0)),
        ],
        out_specs=pl.BlockSpec((BLK, D), lambda i: (i, 0)),
        out_shape=jax.ShapeDtypeStruct((NP, D), jnp.float32),
    )

    FB = 2000
    assert N % FB == 0
    fgrid = N // FB

    def fin_body(agg_ref, hp_ref, deg_ref, b_ref, o_ref):
        degs = deg_ref[0, :, 0] + deg_ref[1, :, 0] + 1.0
        dinv = lax.rsqrt(degs)[:, None]
        o_ref[...] = ((agg_ref[0] + agg_ref[1] + hp_ref[...]) * dinv
                      + b_ref[...])

    fin = pl.pallas_call(
        fin_body,
        grid=(fgrid,),
        in_specs=[
            pl.BlockSpec((NC, FB, D), lambda i: (0, i, 0)),
            pl.BlockSpec((FB, D), lambda i: (i, 0)),
            pl.BlockSpec((NC, FB, DW), lambda i: (0, i, 0)),
            pl.BlockSpec((1, D), lambda i: (0, 0)),
        ],
        out_specs=pl.BlockSpec((FB, D), lambda i: (i, 0)),
        out_shape=jax.ShapeDtypeStruct((N, D), jnp.float32),
    )

    return mm1, mid, fin


@functools.lru_cache(maxsize=4)
def _build(N, E):
    NP = _ceil_to(N, NS * CH)
    assert NP - N >= 1
    K = _ceil_to(-(-E // (CH * NW)), NBUF)
    EP = NW * K * CH
    agg, deg = _make_sc_kernels(NP, K)
    mm1, mid, fin = _make_tc_kernels(N, NP)
    return NP, K, EP, agg, deg, mm1, mid, fin


def kernel(x, edge_index, W1, b1, W2, b2, W3, b3):
    N, _ = x.shape
    E = edge_index.shape[1]
    NP, K, EP, agg, deg_k, mm1, mid, fin = _build(N, E)

    x_p = jnp.pad(x, ((0, NP - N), (0, 0)))
    pad_idx = N + (jnp.arange(EP - E, dtype=jnp.int32) % (NP - N))
    src_p = jnp.concatenate([edge_index[0], pad_idx])
    dst_p = jnp.concatenate([edge_index[1], pad_idx])
    edge3 = jnp.stack([src_p, dst_p]).reshape(2, NW * K, CH)

    deg = deg_k(edge3)
    b1r, b2r, b3r = (b.reshape(1, D) for b in (b1, b2, b3))

    hp1 = mm1(x_p, W1, deg)
    hp2 = mid(agg(hp1, edge3), hp1, deg, b1r, W2)
    hp3 = mid(agg(hp2, edge3), hp2, deg, b2r, W3)
    return fin(agg(hp3, edge3), hp3, deg, b3r)


# speedup vs baseline: 24.7252x; 24.7252x over previous
import functools

import jax
import jax.numpy as jnp
from jax import lax
from jax.experimental import pallas as pl
from jax.experimental.pallas import tpu as pltpu
from jax.experimental.pallas import tpu_sc as plsc

D = 128
NC = 2
NS = 16
NW = NC * NS
CH = 128
NBUF = 2
NPH = 2


def _ceil_to(a, m):
    return (a + m - 1) // m * m


def _splat16(v):
    return lax.broadcast(jnp.float32(v), (16,))


def _make_sc_kernels(NP, K):
    RPT = NP // NS
    KH = K // NPH
    G = KH // NBUF
    assert RPT % CH == 0 and KH % NBUF == 0 and G >= 3
    mesh = plsc.VectorSubcoreMesh(core_axis_name="c", subcore_axis_name="s")

    def agg_body(hp, edge, out, acc, src_v, dst_v, rows_v, *sems):
        gsems = sems[:NBUF]
        ssems = sems[NBUF:]
        c = lax.axis_index("c")
        s = lax.axis_index("s")
        w = c * NS + s

        zero16 = _splat16(0.0)

        def zrow(r, carry):
            for kk in range(D // 16):
                rows_v[0, r, pl.ds(kk * 16, 16)] = zero16
            return carry
        lax.fori_loop(0, CH, zrow, 0)
        row0 = s * RPT
        for i in range(RPT // CH):
            pltpu.sync_copy(rows_v.at[0], acc.at[pl.ds(row0 + i * CH, CH), :])
        plsc.subcore_barrier()

        def gather(t, b):
            pltpu.async_copy(hp.at[src_v.at[t]], rows_v.at[b], gsems[b])

        def gather_wait(t, b):
            pltpu.make_async_copy(hp.at[src_v.at[t]], rows_v.at[b],
                                  gsems[b]).wait()

        def scatter(t, b):
            pltpu.async_copy(rows_v.at[b], acc.at[dst_v.at[t]], ssems[b],
                             add=True)

        def scatter_wait(b):
            pltpu.make_async_copy(rows_v.at[b], acc.at[dst_v.at[0]],
                                  ssems[b]).wait()

        for p in range(NPH):
            pltpu.sync_copy(edge.at[0, w, pl.ds(p * KH, KH), :], src_v)
            pltpu.sync_copy(edge.at[1, w, pl.ds(p * KH, KH), :], dst_v)

            for b in range(NBUF - 1):
                gather(b, b)

            for b in range(NBUF):
                gather_wait(b, b)
                scatter(b, b)
                bn = (b + NBUF - 1) % NBUF
                if b >= 1:
                    scatter_wait(bn)
                gather(b + NBUF - 1, bn)

            def group(g, carry):
                for b in range(NBUF):
                    t = g * NBUF + b
                    gather_wait(t, b)
                    scatter(t, b)
                    bn = (b + NBUF - 1) % NBUF
                    scatter_wait(bn)
                    gather(t + NBUF - 1, bn)
                return carry
            lax.fori_loop(1, G - 1, group, 0)

            t0 = (G - 1) * NBUF
            for b in range(NBUF):
                gather_wait(t0 + b, b)
                scatter(t0 + b, b)
                if b == 0:
                    scatter_wait(NBUF - 1)
                    gather(t0 + NBUF - 1, NBUF - 1)

            for b in range(NBUF):
                scatter_wait(b)

        plsc.subcore_barrier()
        pltpu.sync_copy(acc.at[pl.ds(row0, RPT), :],
                        out.at[c, pl.ds(row0, RPT), :])

    agg = pl.kernel(
        agg_body,
        out_type=jax.ShapeDtypeStruct((NC, NP, D), jnp.float32),
        mesh=mesh,
        scratch_types=[
            pltpu.VMEM_SHARED((NP, D), jnp.float32),
            pltpu.VMEM((KH, CH), jnp.int32),
            pltpu.VMEM((KH, CH), jnp.int32),
            pltpu.VMEM((NBUF, CH, D), jnp.float32),
        ] + [pltpu.SemaphoreType.DMA] * (2 * NBUF),
    )

    def deg_body(edge, out, dst_v, hist):
        c = lax.axis_index("c")
        s = lax.axis_index("s")
        w = c * NS + s
        pltpu.sync_copy(edge.at[1, w], dst_v)
        zero16 = _splat16(0.0)
        one16 = _splat16(1.0)

        def zgrp(r, carry):
            hist[pl.ds(pl.multiple_of(r * 16, 16), 16)] = zero16
            return carry
        lax.fori_loop(0, NP // 16, zgrp, 0)

        def agrp(t, carry):
            for g in range(CH // 16):
                idxv = dst_v[t, pl.ds(g * 16, 16)]
                plsc.addupdate_scatter(hist, [idxv], one16)
            return carry
        lax.fori_loop(0, K, agrp, 0)
        pltpu.sync_copy(hist, out.at[pl.ds(pl.multiple_of(w * NP, 8), NP)])

    deg = pl.kernel(
        deg_body,
        out_type=jax.ShapeDtypeStruct((NW * NP,), jnp.float32),
        mesh=mesh,
        scratch_types=[
            pltpu.VMEM((K, CH), jnp.int32),
            pltpu.VMEM((NP,), jnp.float32),
        ],
        compiler_params=pltpu.CompilerParams(needs_layout_passes=False),
    )

    return agg, deg


def _dinv_block(deg_ref, blk, row0, n):
    degs = jnp.sum(deg_ref[...], axis=0) + 1.0
    dinv = lax.rsqrt(degs)
    rows = lax.broadcasted_iota(jnp.int32, (blk,), 0) + row0
    return jnp.where(rows < n, dinv, 0.0)[:, None]


def _make_tc_kernels(N, NP):
    BLK = 1024
    assert NP % BLK == 0
    grid = NP // BLK

    def mm1_body(x_ref, w_ref, deg_ref, o_ref):
        i = pl.program_id(0)
        dinv = _dinv_block(deg_ref, BLK, i * BLK, N)
        o_ref[...] = jnp.dot(x_ref[...], w_ref[...],
                             preferred_element_type=jnp.float32) * dinv

    mm1 = pl.pallas_call(
        mm1_body,
        grid=(grid,),
        in_specs=[
            pl.BlockSpec((BLK, D), lambda i: (i, 0)),
            pl.BlockSpec((D, D), lambda i: (0, 0)),
            pl.BlockSpec((NW, BLK), lambda i: (0, i)),
        ],
        out_specs=pl.BlockSpec((BLK, D), lambda i: (i, 0)),
        out_shape=jax.ShapeDtypeStruct((NP, D), jnp.float32),
    )

    def mid_body(agg_ref, hp_ref, deg_ref, b_ref, w_ref, o_ref):
        i = pl.program_id(0)
        dinv = _dinv_block(deg_ref, BLK, i * BLK, N)
        u = (agg_ref[0] + agg_ref[1] + hp_ref[...]) * dinv + b_ref[...]
        nrm = jnp.sqrt(jnp.sum(u * u, axis=1, keepdims=True))
        v = jnp.maximum(u / jnp.maximum(nrm, 1e-12), 0.0)
        o_ref[...] = jnp.dot(v, w_ref[...],
                             preferred_element_type=jnp.float32) * dinv

    mid = pl.pallas_call(
        mid_body,
        grid=(grid,),
        in_specs=[
            pl.BlockSpec((NC, BLK, D), lambda i: (0, i, 0)),
            pl.BlockSpec((BLK, D), lambda i: (i, 0)),
            pl.BlockSpec((NW, BLK), lambda i: (0, i)),
            pl.BlockSpec((1, D), lambda i: (0, 0)),
            pl.BlockSpec((D, D), lambda i: (0, 0)),
        ],
        out_specs=pl.BlockSpec((BLK, D), lambda i: (i, 0)),
        out_shape=jax.ShapeDtypeStruct((NP, D), jnp.float32),
    )

    def fin_body(agg_ref, hp_ref, deg_ref, b_ref, o_ref):
        degs = jnp.sum(deg_ref[...], axis=0) + 1.0
        dinv = lax.rsqrt(degs)[:, None]
        o_ref[...] = ((agg_ref[0] + agg_ref[1] + hp_ref[...]) * dinv
                      + b_ref[...])

    fin = pl.pallas_call(
        fin_body,
        grid=(grid,),
        in_specs=[
            pl.BlockSpec((NC, BLK, D), lambda i: (0, i, 0)),
            pl.BlockSpec((BLK, D), lambda i: (i, 0)),
            pl.BlockSpec((NW, BLK), lambda i: (0, i)),
            pl.BlockSpec((1, D), lambda i: (0, 0)),
        ],
        out_specs=pl.BlockSpec((BLK, D), lambda i: (i, 0)),
        out_shape=jax.ShapeDtypeStruct((NP, D), jnp.float32),
    )

    return mm1, mid, fin


@functools.lru_cache(maxsize=4)
def _build(N, E):
    NP = _ceil_to(N, NS * CH)
    assert NP - N >= 1
    K = _ceil_to(-(-E // (CH * NW)), NPH * NBUF)
    EP = NW * K * CH
    agg, deg = _make_sc_kernels(NP, K)
    mm1, mid, fin = _make_tc_kernels(N, NP)
    return NP, K, EP, agg, deg, mm1, mid, fin


def kernel(x, edge_index, W1, b1, W2, b2, W3, b3):
    N, _ = x.shape
    E = edge_index.shape[1]
    NP, K, EP, agg, deg_k, mm1, mid, fin = _build(N, E)

    x_p = jnp.pad(x, ((0, NP - N), (0, 0)))
    pad_idx = N + (jnp.arange(EP - E, dtype=jnp.int32) % (NP - N))
    src_p = jnp.concatenate([edge_index[0], pad_idx])
    dst_p = jnp.concatenate([edge_index[1], pad_idx])
    edge3 = jnp.stack([src_p, dst_p]).reshape(2, NW, K, CH)

    deg = deg_k(edge3).reshape(NW, NP)
    b1r, b2r, b3r = (b.reshape(1, D) for b in (b1, b2, b3))

    hp1 = mm1(x_p, W1, deg)
    hp2 = mid(agg(hp1, edge3), hp1, deg, b1r, W2)
    hp3 = mid(agg(hp2, edge3), hp2, deg, b2r, W3)
    return fin(agg(hp3, edge3), hp3, deg, b3r)[:N]
